# baseline (speedup 1.0000x reference)
import functools

import jax
import jax.numpy as jnp
from jax import lax
from jax.experimental import pallas as pl
from jax.experimental.pallas import tpu as pltpu
from jax.experimental.pallas import tpu_sc as plsc

NUM_CHARS = 96
MAXW = 11
MAXL = MAXW + 1
BSZ, SEQ, NUM_WORDS = 4, 8, 1024
NBS = BSZ * SEQ
NWORKERS = 32
WPW = NUM_WORDS // NWORKERS
LANES = 16
GROUPS = WPW // LANES
NTASK = NBS * GROUPS
D = MAXW + 2
RS = D * LANES


def _dl_body(x_hbm, wt_hbm, wl_hbm, out_hbm, x_v, wt_v, wl_v, swl_v, dtab,
             res_v):
    wid = lax.axis_index("s") * 2 + lax.axis_index("c")
    pltpu.sync_copy(x_hbm, x_v.at[pl.ds(0, NBS * MAXL)])
    pltpu.sync_copy(wt_hbm.at[pl.ds(wid * WPW, WPW)], wt_v)
    pltpu.sync_copy(wl_hbm.at[pl.ds(wid * WPW, WPW)], wl_v)
    lanes = lax.broadcasted_iota(jnp.int32, (LANES,), 0)
    zi = jnp.zeros((LANES,), jnp.int32)
    lanesj = [lanes + LANES * j for j in range(MAXL)]

    for h in range(NBS // LANES):
        base = (lanes + h * LANES) * MAXL
        m = plsc.load_gather(x_v, [base])
        am = zi
        for tt in range(1, MAXL):
            v = plsc.load_gather(x_v, [base + tt])
            better = v > m
            am = jnp.where(better, jnp.int32(tt), am)
            m = jnp.where(better, v, m)
        swl_v[pl.ds(h * LANES, LANES)] = am

    big = jnp.full((LANES,), 1e9, jnp.float32)
    mtwo = jnp.full((LANES,), -2.0, jnp.float32)
    for r in range(D):
        dtab[pl.ds(r * RS, LANES)] = big
    for c in range(1, D):
        dtab[pl.ds(c * LANES, LANES)] = big
    for r in range(1, D):
        dtab[pl.ds(r * RS + LANES, LANES)] = mtwo

    def task_body(t, carry):
        g = t // NBS
        bs = t - g * NBS
        goff = g * LANES
        goffl = lanes + goff
        wl_vec = wl_v[pl.ds(goff, LANES)]
        wch = [plsc.load_gather(wt_v, [goffl, jnp.full((LANES,), jj)])
               for jj in range(MAXW)]
        wlf = wl_vec.astype(jnp.float32)
        outl = (wl_vec + 1) * LANES + lanes

        @pl.when(bs == 0)
        def _():
            for c in range(2, D):
                dtab[pl.ds(RS + c * LANES, LANES)] = jnp.where(
                    c - 1 <= wl_vec, jnp.float32(-2), jnp.float32(-(c + 1)))

        swl_vec = plsc.load_gather(swl_v, [jnp.full((LANES,), bs)])
        swl_s = swl_vec[0]
        maxdist = wlf + swl_vec.astype(jnp.float32)
        xbase = bs * MAXL - 1

        def row_body(i, kd16):
            kd16 = list(kd16)
            xcv = plsc.load_gather(x_v, [jnp.full((LANES,), xbase + i)])
            prev = mtwo
            row = i * RS
            idv = jnp.full((LANES,), row)
            topleft = mtwo
            dbil = lanes
            for j in range(1, MAXL):
                top = dtab[pl.ds(row + (j + 1) * LANES, LANES)]
                meq = wch[j - 1] == xcv
                dt = plsc.load_gather(dtab, [kd16[j] + dbil])
                c3 = topleft + jnp.where(meq, jnp.float32(-2),
                                         jnp.float32(-1))
                val = jnp.minimum(jnp.minimum(jnp.minimum(top, c3),
                                              dt - 3.0), prev)
                dtab[pl.ds(row + RS + (j + 1) * LANES, LANES)] = val
                kd16[j] = jnp.where(meq, idv, kd16[j])
                dbil = jnp.where(meq, lanesj[j], dbil)
                prev = val
                topleft = top
            return tuple(kd16)

        lax.fori_loop(1, swl_s + 1, row_body, tuple([zi] * MAXL))

        outv = plsc.load_gather(dtab, [(swl_vec + 1) * RS + outl])
        res_v[bs, pl.ds(goff, LANES)] = outv + maxdist + 2.0
        return carry

    lax.fori_loop(0, NTASK, task_body, 0)

    pltpu.sync_copy(res_v, out_hbm.at[:, pl.ds(wid * WPW, WPW)])


@functools.lru_cache(maxsize=1)
def _build():
    mesh = plsc.VectorSubcoreMesh(
        core_axis_name="c", subcore_axis_name="s", num_cores=2, num_subcores=16)
    return pl.kernel(
        _dl_body,
        out_type=jax.ShapeDtypeStruct((NBS, NUM_WORDS), jnp.float32),
        mesh=mesh,
        scratch_types=[
            pltpu.VMEM((NBS * MAXL + LANES,), jnp.int32),
            pltpu.VMEM((WPW, MAXW), jnp.int32),
            pltpu.VMEM((WPW,), jnp.int32),
            pltpu.VMEM((NBS,), jnp.int32),
            pltpu.VMEM((D * D * LANES,), jnp.float32),
            pltpu.VMEM((NBS, WPW), jnp.float32),
        ],
        compiler_params=pltpu.CompilerParams(
            needs_layout_passes=False, use_tc_tiling_on_sc=False),
    )


def kernel(x, words, word_lengths):
    out = _build()(x.reshape(-1), words, word_lengths)
    return out.reshape(BSZ, SEQ, NUM_WORDS)


# speedup vs baseline: 1.2592x; 1.2592x over previous
import functools

import jax
import jax.numpy as jnp
from jax import lax
from jax.experimental import pallas as pl
from jax.experimental.pallas import tpu as pltpu
from jax.experimental.pallas import tpu_sc as plsc

NUM_CHARS = 96
MAXW = 11
MAXL = MAXW + 1
BSZ, SEQ, NUM_WORDS = 4, 8, 1024
NBS = BSZ * SEQ
NWORKERS = 32
WPW = NUM_WORDS // NWORKERS
LANES = 16
GROUPS = WPW // LANES
D = MAXW + 2
RS = D * LANES


def _dl_body(x_hbm, wt_hbm, wl_hbm, out_hbm, x_v, wt_v, wl_v, swl_v, dtab0,
             dtab1, res_v):
    wid = lax.axis_index("s") * 2 + lax.axis_index("c")
    pltpu.sync_copy(x_hbm, x_v.at[pl.ds(0, NBS * MAXL)])
    pltpu.sync_copy(wt_hbm.at[pl.ds(wid * WPW * MAXW, WPW * MAXW)], wt_v)
    pltpu.sync_copy(wl_hbm.at[pl.ds(wid * WPW, WPW)], wl_v)
    lanes = lax.broadcasted_iota(jnp.int32, (LANES,), 0)
    zi = jnp.zeros((LANES,), jnp.int32)
    lanesj = [lanes + LANES * j for j in range(MAXL)]
    mask12 = lanes < jnp.int32(MAXL)
    dtabs = (dtab0, dtab1)

    for bs in range(NBS):
        xvecf = jnp.where(mask12,
                          x_v[pl.ds(bs * MAXL, LANES)].astype(jnp.float32),
                          jnp.float32(-1))
        m = jnp.max(xvecf)
        swl_v[bs, :] = plsc.all_reduce_ffs(xvecf == jnp.full((LANES,), m))

    big = jnp.full((LANES,), 1e9, jnp.float32)
    mtwo = jnp.full((LANES,), -2.0, jnp.float32)
    wl_g, wlf_g, outl_g, wchf_g = [], [], [], []
    for g in range(GROUPS):
        dt_g = dtabs[g]
        goff = g * LANES
        wl_vec = wl_v[pl.ds(goff, LANES)]
        wl_g.append(wl_vec)
        wlf_g.append(wl_vec.astype(jnp.float32))
        outl_g.append((wl_vec + 1) * LANES + lanes)
        wchf_g.append((lanes + goff) * MAXW)
        for r in range(D):
            dt_g[pl.ds(r * RS, LANES)] = big
        for c in range(1, D):
            dt_g[pl.ds(c * LANES, LANES)] = big
        for r in range(1, D):
            dt_g[pl.ds(r * RS + LANES, LANES)] = mtwo
        for c in range(2, D):
            dt_g[pl.ds(RS + c * LANES, LANES)] = jnp.where(
                c - 1 <= wl_vec, jnp.float32(-2), jnp.float32(-(c + 1)))

    def task_body(bs, carry):
        swl_vec = swl_v[bs, :]
        swl_s = swl_vec[0]
        swlf = swl_vec.astype(jnp.float32)
        xbase = bs * MAXL - 1

        def row_body(i, kd):
            kd0 = list(kd[0])
            kd1 = list(kd[1])
            xcv = plsc.load_gather(x_v, [jnp.full((LANES,), xbase + i)])
            row = i * RS
            idv = jnp.full((LANES,), row)
            prev0 = mtwo
            prev1 = mtwo
            topleft0 = mtwo
            topleft1 = mtwo
            dbil0 = lanes
            dbil1 = lanes
            for j in range(1, MAXL):
                jv = jnp.full((LANES,), j - 1)
                wch0 = plsc.load_gather(wt_v, [wchf_g[0] + jv])
                wch1 = plsc.load_gather(wt_v, [wchf_g[1] + jv])
                top0 = dtab0[pl.ds(row + (j + 1) * LANES, LANES)]
                top1 = dtab1[pl.ds(row + (j + 1) * LANES, LANES)]
                meq0 = wch0 == xcv
                meq1 = wch1 == xcv
                dt0 = plsc.load_gather(dtab0, [kd0[j] + dbil0])
                dt1 = plsc.load_gather(dtab1, [kd1[j] + dbil1])
                c30 = topleft0 + jnp.where(meq0, jnp.float32(-2),
                                           jnp.float32(-1))
                c31 = topleft1 + jnp.where(meq1, jnp.float32(-2),
                                           jnp.float32(-1))
                val0 = jnp.minimum(jnp.minimum(jnp.minimum(top0, c30),
                                               dt0 - 3.0), prev0)
                val1 = jnp.minimum(jnp.minimum(jnp.minimum(top1, c31),
                                               dt1 - 3.0), prev1)
                dtab0[pl.ds(row + RS + (j + 1) * LANES, LANES)] = val0
                dtab1[pl.ds(row + RS + (j + 1) * LANES, LANES)] = val1
                kd0[j] = jnp.where(meq0, idv, kd0[j])
                kd1[j] = jnp.where(meq1, idv, kd1[j])
                dbil0 = jnp.where(meq0, lanesj[j], dbil0)
                dbil1 = jnp.where(meq1, lanesj[j], dbil1)
                prev0 = val0
                prev1 = val1
                topleft0 = top0
                topleft1 = top1
            return (tuple(kd0), tuple(kd1))

        lax.fori_loop(1, swl_s + 1, row_body,
                      (tuple([zi] * MAXL), tuple([zi] * MAXL)))

        out0 = plsc.load_gather(dtab0, [(swl_vec + 1) * RS + outl_g[0]])
        out1 = plsc.load_gather(dtab1, [(swl_vec + 1) * RS + outl_g[1]])
        res_v[bs, pl.ds(0, LANES)] = out0 + wlf_g[0] + swlf + 2.0
        res_v[bs, pl.ds(LANES, LANES)] = out1 + wlf_g[1] + swlf + 2.0
        return carry

    lax.fori_loop(0, NBS, task_body, 0)

    pltpu.sync_copy(res_v, out_hbm.at[:, pl.ds(wid * WPW, WPW)])


@functools.lru_cache(maxsize=1)
def _build():
    mesh = plsc.VectorSubcoreMesh(
        core_axis_name="c", subcore_axis_name="s", num_cores=2, num_subcores=16)
    return pl.kernel(
        _dl_body,
        out_type=jax.ShapeDtypeStruct((NBS, NUM_WORDS), jnp.float32),
        mesh=mesh,
        scratch_types=[
            pltpu.VMEM((NBS * MAXL + LANES,), jnp.int32),
            pltpu.VMEM((WPW * MAXW,), jnp.int32),
            pltpu.VMEM((WPW,), jnp.int32),
            pltpu.VMEM((NBS, LANES), jnp.int32),
            pltpu.VMEM((D * D * LANES,), jnp.float32),
            pltpu.VMEM((D * D * LANES,), jnp.float32),
            pltpu.VMEM((NBS, WPW), jnp.float32),
        ],
        compiler_params=pltpu.CompilerParams(
            needs_layout_passes=False, use_tc_tiling_on_sc=False),
    )


def kernel(x, words, word_lengths):
    out = _build()(x.reshape(-1), words.reshape(-1), word_lengths)
    return out.reshape(BSZ, SEQ, NUM_WORDS)
